# Initial kernel scaffold; baseline (speedup 1.0000x reference)
import jax
import jax.numpy as jnp
from jax.experimental import pallas as pl


def kernel(x, pos_emb):
    raise NotImplementedError("write your pallas kernel here")


# speedup vs baseline: 1.9420x; 1.9420x over previous
import jax
import jax.numpy as jnp
from jax.experimental import pallas as pl


def _pe_add_body(x_ref, pe_ref, o_ref):
    o_ref[...] = x_ref[...] + pe_ref[...]


def kernel(x, pos_emb):
    B, S, D = x.shape
    SB = 512
    pe = pos_emb[:S]
    return pl.pallas_call(
        _pe_add_body,
        grid=(S // SB,),
        in_specs=[
            pl.BlockSpec((B, SB, D), lambda s: (0, s, 0)),
            pl.BlockSpec((SB, D), lambda s: (s, 0)),
        ],
        out_specs=pl.BlockSpec((B, SB, D), lambda s: (0, s, 0)),
        out_shape=jax.ShapeDtypeStruct((B, S, D), x.dtype),
    )(x, pe)
